# Initial kernel scaffold; baseline (speedup 1.0000x reference)
import jax
import jax.numpy as jnp
from jax.experimental import pallas as pl


def kernel(x, W_router, w_up, b_up, w_down, b_down):
    raise NotImplementedError("write your pallas kernel here")


# speedup vs baseline: 1.2778x; 1.2778x over previous
import functools

import jax
import jax.numpy as jnp
from jax import lax
from jax.experimental import pallas as pl
from jax.experimental.pallas import tpu as pltpu
from jax.experimental.pallas import tpu_sc as plsc

N, D, FF, E = 2048, 768, 1024, 64
T = 32
NC = N // T + E
P = NC * T
NW = 32


def _ffn_body(ce_ref, act_ref, x_ref, wup_ref, bup_ref, wdn_ref, bdn_ref,
              out_ref):
    @pl.when(act_ref[pl.program_id(0)] != 0)
    def _():
        xb = x_ref[...]
        h = lax.dot_general(xb, wup_ref[0],
                            (((1,), (1,)), ((), ())),
                            preferred_element_type=jnp.float32,
                            precision=lax.Precision.HIGHEST)
        h = h + bup_ref[0]
        h = h * jax.nn.sigmoid(h)
        y = lax.dot_general(h, wdn_ref[0],
                            (((1,), (1,)), ((), ())),
                            preferred_element_type=jnp.float32,
                            precision=lax.Precision.HIGHEST)
        out_ref[...] = y + bdn_ref[0]


def _sc_gather(table, idx, nrows):
    b_per_w = nrows // NW
    mesh = plsc.VectorSubcoreMesh(core_axis_name="c", subcore_axis_name="s")

    @functools.partial(
        pl.kernel, mesh=mesh,
        out_type=jax.ShapeDtypeStruct((nrows, D), jnp.float32),
        scratch_types=[
            pltpu.VMEM((b_per_w,), jnp.int32),
            pltpu.VMEM((b_per_w, D), jnp.float32),
            pltpu.SemaphoreType.DMA,
        ],
    )
    def k(table_hbm, idx_hbm, out_hbm, idx_v, rows_v, sem):
        wid = lax.axis_index("s") * 2 + lax.axis_index("c")
        base = wid * b_per_w
        pltpu.sync_copy(idx_hbm.at[pl.ds(base, b_per_w)], idx_v)
        pltpu.async_copy(table_hbm.at[idx_v], rows_v, sem).wait()
        pltpu.sync_copy(rows_v, out_hbm.at[pl.ds(base, b_per_w)])

    return k(table, idx)


def kernel(x, W_router, w_up, b_up, w_down, b_down):
    logits = x @ W_router
    _, top_idx = lax.top_k(logits, 1)
    assign = top_idx[:, 0].astype(jnp.int32)

    order = jnp.argsort(assign).astype(jnp.int32)
    sorted_assign = jnp.take(assign, order)
    counts = jnp.bincount(assign, length=E).astype(jnp.int32)
    starts = jnp.concatenate(
        [jnp.zeros((1,), jnp.int32), jnp.cumsum(counts)[:-1].astype(jnp.int32)])
    cc = (counts + T - 1) // T
    cum = jnp.cumsum(cc).astype(jnp.int32)
    total_chunks = cum[-1]
    chunk_ids = jnp.arange(NC, dtype=jnp.int32)
    chunk_e_raw = jnp.searchsorted(cum, chunk_ids, side="right").astype(jnp.int32)
    last_e = jnp.searchsorted(cum, total_chunks - 1, side="right").astype(jnp.int32)
    active = (chunk_ids < total_chunks).astype(jnp.int32)
    chunk_expert = jnp.where(active == 1, chunk_e_raw, last_e)
    p_start = (cum - cc) * T
    pos_sorted = (jnp.take(p_start, sorted_assign)
                  + jnp.arange(N, dtype=jnp.int32)
                  - jnp.take(starts, sorted_assign))
    src = jnp.zeros((P,), jnp.int32).at[pos_sorted].set(order)
    pos_token = jnp.zeros((N,), jnp.int32).at[order].set(pos_sorted)

    x_padded = _sc_gather(x, src, P)

    b_up3 = b_up.reshape(E, 1, FF)
    b_down3 = b_down.reshape(E, 1, D)
    grid_spec = pltpu.PrefetchScalarGridSpec(
        num_scalar_prefetch=2,
        grid=(NC,),
        in_specs=[
            pl.BlockSpec((T, D), lambda c, ce, act: (c, 0)),
            pl.BlockSpec((1, FF, D), lambda c, ce, act: (ce[c], 0, 0)),
            pl.BlockSpec((1, 1, FF), lambda c, ce, act: (ce[c], 0, 0)),
            pl.BlockSpec((1, D, FF), lambda c, ce, act: (ce[c], 0, 0)),
            pl.BlockSpec((1, 1, D), lambda c, ce, act: (ce[c], 0, 0)),
        ],
        out_specs=pl.BlockSpec((T, D), lambda c, ce, act: (c, 0)),
    )
    y_padded = pl.pallas_call(
        _ffn_body,
        grid_spec=grid_spec,
        out_shape=jax.ShapeDtypeStruct((P, D), jnp.float32),
    )(chunk_expert, active, x_padded, w_up, b_up3, w_down, b_down3)

    return _sc_gather(y_padded, pos_token, N)


# speedup vs baseline: 3.2099x; 2.5121x over previous
import functools

import jax
import jax.numpy as jnp
from jax import lax
from jax.experimental import pallas as pl
from jax.experimental.pallas import tpu as pltpu
from jax.experimental.pallas import tpu_sc as plsc

N, D, FF, E = 2048, 768, 1024, 64
T = 128
NC = N // T + E
P = NC * T
NW = 32


def _ffn_body(ce_ref, act_ref, x_ref, wup_ref, bup_ref, wdn_ref, bdn_ref,
              out_ref):
    @pl.when(act_ref[pl.program_id(0)] != 0)
    def _():
        xb = x_ref[...]
        h = lax.dot_general(xb, wup_ref[0],
                            (((1,), (1,)), ((), ())),
                            preferred_element_type=jnp.float32)
        h = h + bup_ref[0]
        h = h * jax.nn.sigmoid(h)
        y = lax.dot_general(h, wdn_ref[0],
                            (((1,), (1,)), ((), ())),
                            preferred_element_type=jnp.float32)
        out_ref[...] = y + bdn_ref[0]


def _sc_scatter(rows, idx, nrows_out):
    nrows_in = rows.shape[0]
    b_per_w = nrows_in // NW
    mesh = plsc.VectorSubcoreMesh(core_axis_name="c", subcore_axis_name="s")

    @functools.partial(
        pl.kernel, mesh=mesh,
        out_type=jax.ShapeDtypeStruct((nrows_out, D), jnp.float32),
        scratch_types=[
            pltpu.VMEM((b_per_w,), jnp.int32),
            pltpu.VMEM((b_per_w, D), jnp.float32),
            pltpu.SemaphoreType.DMA,
        ],
    )
    def k(rows_hbm, idx_hbm, out_hbm, idx_v, rows_v, sem):
        wid = lax.axis_index("s") * 2 + lax.axis_index("c")
        base = wid * b_per_w
        pltpu.sync_copy(idx_hbm.at[pl.ds(base, b_per_w)], idx_v)
        pltpu.sync_copy(rows_hbm.at[pl.ds(base, b_per_w)], rows_v)
        pltpu.async_copy(rows_v, out_hbm.at[idx_v], sem).wait()

    return k(rows, idx)


def _sc_gather(table, idx, nrows):
    b_per_w = nrows // NW
    mesh = plsc.VectorSubcoreMesh(core_axis_name="c", subcore_axis_name="s")

    @functools.partial(
        pl.kernel, mesh=mesh,
        out_type=jax.ShapeDtypeStruct((nrows, D), jnp.float32),
        scratch_types=[
            pltpu.VMEM((b_per_w,), jnp.int32),
            pltpu.VMEM((b_per_w, D), jnp.float32),
            pltpu.SemaphoreType.DMA,
        ],
    )
    def k(table_hbm, idx_hbm, out_hbm, idx_v, rows_v, sem):
        wid = lax.axis_index("s") * 2 + lax.axis_index("c")
        base = wid * b_per_w
        pltpu.sync_copy(idx_hbm.at[pl.ds(base, b_per_w)], idx_v)
        pltpu.async_copy(table_hbm.at[idx_v], rows_v, sem).wait()
        pltpu.sync_copy(rows_v, out_hbm.at[pl.ds(base, b_per_w)])

    return k(table, idx)


def kernel(x, W_router, w_up, b_up, w_down, b_down):
    logits = x @ W_router
    _, top_idx = lax.top_k(logits, 1)
    assign = top_idx[:, 0].astype(jnp.int32)

    order = jnp.argsort(assign).astype(jnp.int32)
    sorted_assign = jnp.take(assign, order)
    counts = jnp.bincount(assign, length=E).astype(jnp.int32)
    starts = jnp.concatenate(
        [jnp.zeros((1,), jnp.int32), jnp.cumsum(counts)[:-1].astype(jnp.int32)])
    cc = (counts + T - 1) // T
    cum = jnp.cumsum(cc).astype(jnp.int32)
    total_chunks = cum[-1]
    chunk_ids = jnp.arange(NC, dtype=jnp.int32)
    chunk_e_raw = jnp.searchsorted(cum, chunk_ids, side="right").astype(jnp.int32)
    last_e = jnp.searchsorted(cum, total_chunks - 1, side="right").astype(jnp.int32)
    active = (chunk_ids < total_chunks).astype(jnp.int32)
    chunk_expert = jnp.where(active == 1, chunk_e_raw, last_e)
    p_start = (cum - cc) * T
    pos_sorted = (jnp.take(p_start, sorted_assign)
                  + jnp.arange(N, dtype=jnp.int32)
                  - jnp.take(starts, sorted_assign))
    pos_token = jnp.zeros((N,), jnp.int32).at[order].set(pos_sorted)

    x_padded = _sc_scatter(x, pos_token, P)

    b_up3 = b_up.reshape(E, 1, FF)
    b_down3 = b_down.reshape(E, 1, D)
    grid_spec = pltpu.PrefetchScalarGridSpec(
        num_scalar_prefetch=2,
        grid=(NC,),
        in_specs=[
            pl.BlockSpec((T, D), lambda c, ce, act: (c, 0)),
            pl.BlockSpec((1, FF, D), lambda c, ce, act: (ce[c], 0, 0)),
            pl.BlockSpec((1, 1, FF), lambda c, ce, act: (ce[c], 0, 0)),
            pl.BlockSpec((1, D, FF), lambda c, ce, act: (ce[c], 0, 0)),
            pl.BlockSpec((1, 1, D), lambda c, ce, act: (ce[c], 0, 0)),
        ],
        out_specs=pl.BlockSpec((T, D), lambda c, ce, act: (c, 0)),
    )
    y_padded = pl.pallas_call(
        _ffn_body,
        grid_spec=grid_spec,
        out_shape=jax.ShapeDtypeStruct((P, D), jnp.float32),
    )(chunk_expert, active, x_padded, w_up, b_up3, w_down, b_down3)

    return _sc_gather(y_padded, pos_token, N)


# speedup vs baseline: 4.5408x; 1.4146x over previous
import functools

import jax
import jax.numpy as jnp
from jax import lax
from jax.experimental import pallas as pl
from jax.experimental.pallas import tpu as pltpu
from jax.experimental.pallas import tpu_sc as plsc

N, D, FF, E = 2048, 768, 1024, 64
T = 128
NC = N // T + E
P = NC * T
NW = 32


def _ffn_body(ce_ref, act_ref, x_ref, wup_ref, bup_ref, wdn_ref, bdn_ref,
              out_ref):
    @pl.when(act_ref[pl.program_id(0)] != 0)
    def _():
        xb = x_ref[...]
        h = lax.dot_general(xb, wup_ref[0],
                            (((1,), (1,)), ((), ())),
                            preferred_element_type=jnp.float32)
        h = h + bup_ref[0]
        h = h * jax.nn.sigmoid(h)
        y = lax.dot_general(h, wdn_ref[0],
                            (((1,), (1,)), ((), ())),
                            preferred_element_type=jnp.float32)
        out_ref[...] = y + bdn_ref[0]


def _sc_plan(assign):
    mesh = plsc.ScalarSubcoreMesh(axis_name="core", num_cores=2)
    outs = (jax.ShapeDtypeStruct((N,), jnp.int32),
            jax.ShapeDtypeStruct((NC,), jnp.int32),
            jax.ShapeDtypeStruct((NC,), jnp.int32))

    @functools.partial(
        pl.kernel, mesh=mesh, out_type=outs,
        scratch_types=[
            pltpu.SMEM((N,), jnp.int32),
            pltpu.SMEM((N,), jnp.int32),
            pltpu.SMEM((E,), jnp.int32),
            pltpu.SMEM((E,), jnp.int32),
            pltpu.SMEM((NC,), jnp.int32),
            pltpu.SMEM((NC,), jnp.int32),
            pltpu.SMEM((1,), jnp.int32),
            pltpu.SemaphoreType.DMA,
        ],
    )
    def k(assign_hbm, pos_hbm, ce_hbm, act_hbm,
          a_s, pos_s, cnt_s, ps_s, ce_s, act_s, cur_s, sem):
        @pl.when(lax.axis_index("core") == 0)
        def _():
            pltpu.async_copy(assign_hbm, a_s, sem).wait()

            @pl.loop(0, E)
            def _(e):
                cnt_s[e] = 0

            @pl.loop(0, N)
            def _(i):
                e = a_s[i]
                pos_s[i] = cnt_s[e]
                cnt_s[e] = cnt_s[e] + 1

            cur_s[0] = 0

            @pl.loop(0, E)
            def _(e):
                base = cur_s[0]
                ps_s[e] = base * T

                def body(j, _):
                    ce_s[base + j] = e
                    return 0
                lax.fori_loop(0, (cnt_s[e] + (T - 1)) // T, body, 0)
                cur_s[0] = base + (cnt_s[e] + (T - 1)) // T

            total = cur_s[0]
            last_e = ce_s[total - 1]

            @pl.loop(0, NC)
            def _(c):
                act_s[c] = (c < total).astype(jnp.int32)

                @pl.when(c >= total)
                def _():
                    ce_s[c] = last_e

            @pl.loop(0, N)
            def _(i):
                pos_s[i] = pos_s[i] + ps_s[a_s[i]]

            pltpu.async_copy(pos_s, pos_hbm, sem).wait()
            pltpu.async_copy(ce_s, ce_hbm, sem).wait()
            pltpu.async_copy(act_s, act_hbm, sem).wait()

    return k(assign)


def _sc_scatter(rows, idx, nrows_out):
    nrows_in = rows.shape[0]
    b_per_w = nrows_in // NW
    mesh = plsc.VectorSubcoreMesh(core_axis_name="c", subcore_axis_name="s")

    @functools.partial(
        pl.kernel, mesh=mesh,
        out_type=jax.ShapeDtypeStruct((nrows_out, D), jnp.float32),
        scratch_types=[
            pltpu.VMEM((b_per_w,), jnp.int32),
            pltpu.VMEM((b_per_w, D), jnp.float32),
            pltpu.SemaphoreType.DMA,
        ],
    )
    def k(rows_hbm, idx_hbm, out_hbm, idx_v, rows_v, sem):
        wid = lax.axis_index("s") * 2 + lax.axis_index("c")
        base = wid * b_per_w
        pltpu.sync_copy(idx_hbm.at[pl.ds(base, b_per_w)], idx_v)
        pltpu.sync_copy(rows_hbm.at[pl.ds(base, b_per_w)], rows_v)
        pltpu.async_copy(rows_v, out_hbm.at[idx_v], sem).wait()

    return k(rows, idx)


def _sc_gather(table, idx, nrows):
    b_per_w = nrows // NW
    mesh = plsc.VectorSubcoreMesh(core_axis_name="c", subcore_axis_name="s")

    @functools.partial(
        pl.kernel, mesh=mesh,
        out_type=jax.ShapeDtypeStruct((nrows, D), jnp.float32),
        scratch_types=[
            pltpu.VMEM((b_per_w,), jnp.int32),
            pltpu.VMEM((b_per_w, D), jnp.float32),
            pltpu.SemaphoreType.DMA,
        ],
    )
    def k(table_hbm, idx_hbm, out_hbm, idx_v, rows_v, sem):
        wid = lax.axis_index("s") * 2 + lax.axis_index("c")
        base = wid * b_per_w
        pltpu.sync_copy(idx_hbm.at[pl.ds(base, b_per_w)], idx_v)
        pltpu.async_copy(table_hbm.at[idx_v], rows_v, sem).wait()
        pltpu.sync_copy(rows_v, out_hbm.at[pl.ds(base, b_per_w)])

    return k(table, idx)


def kernel(x, W_router, w_up, b_up, w_down, b_down):
    logits = x @ W_router
    _, top_idx = lax.top_k(logits, 1)
    assign = top_idx[:, 0].astype(jnp.int32)

    pos_token, chunk_expert, active = _sc_plan(assign)

    x_padded = _sc_scatter(x, pos_token, P)

    b_up3 = b_up.reshape(E, 1, FF)
    b_down3 = b_down.reshape(E, 1, D)
    grid_spec = pltpu.PrefetchScalarGridSpec(
        num_scalar_prefetch=2,
        grid=(NC,),
        in_specs=[
            pl.BlockSpec((T, D), lambda c, ce, act: (c, 0)),
            pl.BlockSpec((1, FF, D), lambda c, ce, act: (ce[c], 0, 0)),
            pl.BlockSpec((1, 1, FF), lambda c, ce, act: (ce[c], 0, 0)),
            pl.BlockSpec((1, D, FF), lambda c, ce, act: (ce[c], 0, 0)),
            pl.BlockSpec((1, 1, D), lambda c, ce, act: (ce[c], 0, 0)),
        ],
        out_specs=pl.BlockSpec((T, D), lambda c, ce, act: (c, 0)),
    )
    y_padded = pl.pallas_call(
        _ffn_body,
        grid_spec=grid_spec,
        out_shape=jax.ShapeDtypeStruct((P, D), jnp.float32),
    )(chunk_expert, active, x_padded, w_up, b_up3, w_down, b_down3)

    return _sc_gather(y_padded, pos_token, N)


# speedup vs baseline: 4.5523x; 1.0025x over previous
import functools

import jax
import jax.numpy as jnp
from jax import lax
from jax.experimental import pallas as pl
from jax.experimental.pallas import tpu as pltpu
from jax.experimental.pallas import tpu_sc as plsc

N, D, FF, E = 2048, 768, 1024, 64
T = 64
NC = N // T + E
P = NC * T
NW = 32


def _ffn_body(ce_ref, act_ref, x_ref, wup_ref, bup_ref, wdn_ref, bdn_ref,
              out_ref):
    @pl.when(act_ref[pl.program_id(0)] != 0)
    def _():
        xb = x_ref[...]
        h = lax.dot_general(xb, wup_ref[0],
                            (((1,), (1,)), ((), ())),
                            preferred_element_type=jnp.float32)
        h = h + bup_ref[0]
        h = h * jax.nn.sigmoid(h)
        y = lax.dot_general(h, wdn_ref[0],
                            (((1,), (1,)), ((), ())),
                            preferred_element_type=jnp.float32)
        out_ref[...] = y + bdn_ref[0]


def _sc_plan(assign):
    mesh = plsc.ScalarSubcoreMesh(axis_name="core", num_cores=2)
    outs = (jax.ShapeDtypeStruct((N,), jnp.int32),
            jax.ShapeDtypeStruct((NC,), jnp.int32),
            jax.ShapeDtypeStruct((NC,), jnp.int32))

    @functools.partial(
        pl.kernel, mesh=mesh, out_type=outs,
        scratch_types=[
            pltpu.SMEM((N,), jnp.int32),
            pltpu.SMEM((N,), jnp.int32),
            pltpu.SMEM((E,), jnp.int32),
            pltpu.SMEM((E,), jnp.int32),
            pltpu.SMEM((NC,), jnp.int32),
            pltpu.SMEM((NC,), jnp.int32),
            pltpu.SMEM((1,), jnp.int32),
            pltpu.SemaphoreType.DMA,
        ],
    )
    def k(assign_hbm, pos_hbm, ce_hbm, act_hbm,
          a_s, pos_s, cnt_s, ps_s, ce_s, act_s, cur_s, sem):
        @pl.when(lax.axis_index("core") == 0)
        def _():
            pltpu.async_copy(assign_hbm, a_s, sem).wait()

            @pl.loop(0, E)
            def _(e):
                cnt_s[e] = 0

            @pl.loop(0, N)
            def _(i):
                e = a_s[i]
                pos_s[i] = cnt_s[e]
                cnt_s[e] = cnt_s[e] + 1

            cur_s[0] = 0

            @pl.loop(0, E)
            def _(e):
                base = cur_s[0]
                ps_s[e] = base * T

                def body(j, _):
                    ce_s[base + j] = e
                    return 0
                lax.fori_loop(0, (cnt_s[e] + (T - 1)) // T, body, 0)
                cur_s[0] = base + (cnt_s[e] + (T - 1)) // T

            total = cur_s[0]
            last_e = ce_s[total - 1]

            @pl.loop(0, NC)
            def _(c):
                act_s[c] = (c < total).astype(jnp.int32)

                @pl.when(c >= total)
                def _():
                    ce_s[c] = last_e

            @pl.loop(0, N)
            def _(i):
                pos_s[i] = pos_s[i] + ps_s[a_s[i]]

            pltpu.async_copy(pos_s, pos_hbm, sem).wait()
            pltpu.async_copy(ce_s, ce_hbm, sem).wait()
            pltpu.async_copy(act_s, act_hbm, sem).wait()

    return k(assign)


def _sc_scatter(rows, idx, nrows_out):
    nrows_in = rows.shape[0]
    b_per_w = nrows_in // NW
    mesh = plsc.VectorSubcoreMesh(core_axis_name="c", subcore_axis_name="s")

    @functools.partial(
        pl.kernel, mesh=mesh,
        out_type=jax.ShapeDtypeStruct((nrows_out, D), jnp.float32),
        scratch_types=[
            pltpu.VMEM((b_per_w,), jnp.int32),
            pltpu.VMEM((b_per_w, D), jnp.float32),
            pltpu.SemaphoreType.DMA,
        ],
    )
    def k(rows_hbm, idx_hbm, out_hbm, idx_v, rows_v, sem):
        wid = lax.axis_index("s") * 2 + lax.axis_index("c")
        base = wid * b_per_w
        pltpu.sync_copy(idx_hbm.at[pl.ds(base, b_per_w)], idx_v)
        pltpu.sync_copy(rows_hbm.at[pl.ds(base, b_per_w)], rows_v)
        pltpu.async_copy(rows_v, out_hbm.at[idx_v], sem).wait()

    return k(rows, idx)


def _sc_gather(table, idx, nrows):
    b_per_w = nrows // NW
    mesh = plsc.VectorSubcoreMesh(core_axis_name="c", subcore_axis_name="s")

    @functools.partial(
        pl.kernel, mesh=mesh,
        out_type=jax.ShapeDtypeStruct((nrows, D), jnp.float32),
        scratch_types=[
            pltpu.VMEM((b_per_w,), jnp.int32),
            pltpu.VMEM((b_per_w, D), jnp.float32),
            pltpu.SemaphoreType.DMA,
        ],
    )
    def k(table_hbm, idx_hbm, out_hbm, idx_v, rows_v, sem):
        wid = lax.axis_index("s") * 2 + lax.axis_index("c")
        base = wid * b_per_w
        pltpu.sync_copy(idx_hbm.at[pl.ds(base, b_per_w)], idx_v)
        pltpu.async_copy(table_hbm.at[idx_v], rows_v, sem).wait()
        pltpu.sync_copy(rows_v, out_hbm.at[pl.ds(base, b_per_w)])

    return k(table, idx)


def _ffn_call(chunk_expert, active, x_padded, w_up, b_up, w_down, b_down):
    b_up3 = b_up.reshape(E, 1, FF)
    b_down3 = b_down.reshape(E, 1, D)
    grid_spec = pltpu.PrefetchScalarGridSpec(
        num_scalar_prefetch=2,
        grid=(NC,),
        in_specs=[
            pl.BlockSpec((T, D), lambda c, ce, act: (c, 0)),
            pl.BlockSpec((1, FF, D), lambda c, ce, act: (ce[c], 0, 0)),
            pl.BlockSpec((1, 1, FF), lambda c, ce, act: (ce[c], 0, 0)),
            pl.BlockSpec((1, D, FF), lambda c, ce, act: (ce[c], 0, 0)),
            pl.BlockSpec((1, 1, D), lambda c, ce, act: (ce[c], 0, 0)),
        ],
        out_specs=pl.BlockSpec((T, D), lambda c, ce, act: (c, 0)),
    )
    return pl.pallas_call(
        _ffn_body,
        grid_spec=grid_spec,
        out_shape=jax.ShapeDtypeStruct((P, D), jnp.float32),
    )(chunk_expert, active, x_padded, w_up, b_up3, w_down, b_down3)


def kernel(x, W_router, w_up, b_up, w_down, b_down):
    logits = x @ W_router
    _, top_idx = lax.top_k(logits, 1)
    assign = top_idx[:, 0].astype(jnp.int32)

    pos_token, chunk_expert, active = _sc_plan(assign)

    x_padded = _sc_scatter(x, pos_token, P)

    y_padded = _ffn_call(chunk_expert, active, x_padded, w_up, b_up, w_down, b_down)

    return _sc_gather(y_padded, pos_token, N)


# speedup vs baseline: 5.0527x; 1.1099x over previous
import dataclasses
import functools

import jax
import jax.numpy as jnp
from jax import lax
from jax.experimental import pallas as pl
from jax.experimental.pallas import tpu as pltpu
from jax.experimental.pallas import tpu_sc as plsc

N, D, FF, E = 2048, 768, 1024, 64
T = 64
NC = N // T + E
P = NC * T
NW = 32


def _ffn_body(ce_ref, act_ref, x_ref, wup_ref, bup_ref, wdn_ref, bdn_ref,
              out_ref):
    @pl.when(act_ref[pl.program_id(0)] != 0)
    def _():
        xb = x_ref[...]
        h = lax.dot_general(xb, wup_ref[0],
                            (((1,), (1,)), ((), ())),
                            preferred_element_type=jnp.float32)
        h = h + bup_ref[0]
        h = h * jax.nn.sigmoid(h)
        y = lax.dot_general(h, wdn_ref[0],
                            (((1,), (1,)), ((), ())),
                            preferred_element_type=jnp.float32)
        out_ref[...] = y + bdn_ref[0]


_SC_CP = pltpu.CompilerParams()
if "needs_layout_passes" in pltpu.CompilerParams.__dataclass_fields__:
    _SC_CP = dataclasses.replace(_SC_CP, needs_layout_passes=False)

L = 16


def _sc_plan(assign):
    mesh = plsc.VectorSubcoreMesh(core_axis_name="c", subcore_axis_name="s")
    outs = (jax.ShapeDtypeStruct((N,), jnp.int32),
            jax.ShapeDtypeStruct((NC,), jnp.int32),
            jax.ShapeDtypeStruct((NC,), jnp.int32))

    @functools.partial(
        pl.kernel, mesh=mesh, out_type=outs, compiler_params=_SC_CP,
        scratch_types=[
            pltpu.VMEM((N,), jnp.int32),
            pltpu.VMEM((N,), jnp.int32),
            pltpu.VMEM((E,), jnp.int32),
            pltpu.VMEM((E,), jnp.int32),
            pltpu.VMEM((E,), jnp.int32),
            pltpu.VMEM((NC,), jnp.int32),
            pltpu.VMEM((NC,), jnp.int32),
            pltpu.SemaphoreType.DMA,
        ],
    )
    def k(assign_hbm, pos_hbm, ce_hbm, act_hbm,
          a_v, pos_v, cnt_v, ps_v, cum_v, ce_v, act_v, sem):
        wid = lax.axis_index("s") * 2 + lax.axis_index("c")

        @pl.when(wid == 0)
        def _():
            pltpu.async_copy(assign_hbm, a_v, sem).wait()
            zeros = jnp.zeros((L,), jnp.int32)

            @pl.loop(0, E, step=L)
            def _(e):
                cnt_v[pl.ds(e, L)] = zeros

            @pl.loop(0, N, step=L)
            def _(i):
                v = a_v[pl.ds(i, L)]
                g = plsc.load_gather(cnt_v, [v])
                occ, is_last = plsc.scan_count(v)
                pos_v[pl.ds(i, L)] = g + occ - 1
                plsc.store_scatter(cnt_v, [v], g + occ, mask=is_last)

            carry = jnp.int32(0)
            for eb in range(0, E, L):
                cnt = cnt_v[pl.ds(eb, L)]
                cc = (cnt + (T - 1)) // T
                cum = jnp.cumsum(cc) + carry
                carry = cum[L - 1]
                cum_v[pl.ds(eb, L)] = cum
                ps_v[pl.ds(eb, L)] = (cum - cc) * T
            total = carry

            for cb in range(0, NC, L):
                ce_v[pl.ds(cb, L)] = zeros
            for eb in range(0, E, L):
                cnt = cnt_v[pl.ds(eb, L)]
                cc = (cnt + (T - 1)) // T
                cum = cum_v[pl.ds(eb, L)]
                e_ids = eb + lax.iota(jnp.int32, L)
                plsc.store_scatter(ce_v, [cum - cc], e_ids, mask=cc > 0)
            carry = jnp.int32(0)
            for cb in range(0, NC, L):
                cid = cb + lax.iota(jnp.int32, L)
                ce = jnp.maximum(plsc.cummax(ce_v[pl.ds(cb, L)]), carry)
                carry = ce[L - 1]
                ce_v[pl.ds(cb, L)] = ce
                act_v[pl.ds(cb, L)] = (cid < total).astype(jnp.int32)

            @pl.loop(0, N, step=L)
            def _(i):
                v = a_v[pl.ds(i, L)]
                pos_v[pl.ds(i, L)] = (pos_v[pl.ds(i, L)]
                                      + plsc.load_gather(ps_v, [v]))

            pltpu.async_copy(pos_v, pos_hbm, sem).wait()
            pltpu.async_copy(ce_v, ce_hbm, sem).wait()
            pltpu.async_copy(act_v, act_hbm, sem).wait()

    return k(assign)


def _sc_scatter(rows, idx, nrows_out):
    nrows_in = rows.shape[0]
    b_per_w = nrows_in // NW
    mesh = plsc.VectorSubcoreMesh(core_axis_name="c", subcore_axis_name="s")

    @functools.partial(
        pl.kernel, mesh=mesh,
        out_type=jax.ShapeDtypeStruct((nrows_out, D), jnp.float32),
        scratch_types=[
            pltpu.VMEM((b_per_w,), jnp.int32),
            pltpu.VMEM((b_per_w, D), jnp.float32),
            pltpu.SemaphoreType.DMA,
        ],
    )
    def k(rows_hbm, idx_hbm, out_hbm, idx_v, rows_v, sem):
        wid = lax.axis_index("s") * 2 + lax.axis_index("c")
        base = wid * b_per_w
        pltpu.sync_copy(idx_hbm.at[pl.ds(base, b_per_w)], idx_v)
        pltpu.sync_copy(rows_hbm.at[pl.ds(base, b_per_w)], rows_v)
        pltpu.async_copy(rows_v, out_hbm.at[idx_v], sem).wait()

    return k(rows, idx)


def _sc_gather(table, idx, nrows):
    b_per_w = nrows // NW
    mesh = plsc.VectorSubcoreMesh(core_axis_name="c", subcore_axis_name="s")

    @functools.partial(
        pl.kernel, mesh=mesh,
        out_type=jax.ShapeDtypeStruct((nrows, D), jnp.float32),
        scratch_types=[
            pltpu.VMEM((b_per_w,), jnp.int32),
            pltpu.VMEM((b_per_w, D), jnp.float32),
            pltpu.SemaphoreType.DMA,
        ],
    )
    def k(table_hbm, idx_hbm, out_hbm, idx_v, rows_v, sem):
        wid = lax.axis_index("s") * 2 + lax.axis_index("c")
        base = wid * b_per_w
        pltpu.sync_copy(idx_hbm.at[pl.ds(base, b_per_w)], idx_v)
        pltpu.async_copy(table_hbm.at[idx_v], rows_v, sem).wait()
        pltpu.sync_copy(rows_v, out_hbm.at[pl.ds(base, b_per_w)])

    return k(table, idx)


def _ffn_call(chunk_expert, active, x_padded, w_up, b_up, w_down, b_down):
    b_up3 = b_up.reshape(E, 1, FF)
    b_down3 = b_down.reshape(E, 1, D)
    grid_spec = pltpu.PrefetchScalarGridSpec(
        num_scalar_prefetch=2,
        grid=(NC,),
        in_specs=[
            pl.BlockSpec((T, D), lambda c, ce, act: (c, 0)),
            pl.BlockSpec((1, FF, D), lambda c, ce, act: (ce[c], 0, 0)),
            pl.BlockSpec((1, 1, FF), lambda c, ce, act: (ce[c], 0, 0)),
            pl.BlockSpec((1, D, FF), lambda c, ce, act: (ce[c], 0, 0)),
            pl.BlockSpec((1, 1, D), lambda c, ce, act: (ce[c], 0, 0)),
        ],
        out_specs=pl.BlockSpec((T, D), lambda c, ce, act: (c, 0)),
    )
    return pl.pallas_call(
        _ffn_body,
        grid_spec=grid_spec,
        out_shape=jax.ShapeDtypeStruct((P, D), jnp.float32),
    )(chunk_expert, active, x_padded, w_up, b_up3, w_down, b_down3)


def kernel(x, W_router, w_up, b_up, w_down, b_down):
    logits = x @ W_router
    _, top_idx = lax.top_k(logits, 1)
    assign = top_idx[:, 0].astype(jnp.int32)

    pos_token, chunk_expert, active = _sc_plan(assign)

    x_padded = _sc_scatter(x, pos_token, P)

    y_padded = _ffn_call(chunk_expert, active, x_padded, w_up, b_up, w_down, b_down)

    return _sc_gather(y_padded, pos_token, N)
